# Initial kernel scaffold; baseline (speedup 1.0000x reference)
import jax
import jax.numpy as jnp
from jax.experimental import pallas as pl


def kernel(x, emb_weight):
    raise NotImplementedError("write your pallas kernel here")


# speedup vs baseline: 2.2439x; 2.2439x over previous
import math
import functools

import jax
import jax.numpy as jnp
from jax import lax
from jax.experimental import pallas as pl
from jax.experimental.pallas import tpu as pltpu
from jax.experimental.pallas import tpu_sc as plsc

NUM_EMBEDDINGS = 1000000
EMBED_DIM = 64
SEQ = 200
BATCH = 4096

NC = 2
NS = 16
NW = NC * NS

B_TOTAL = BATCH * SEQ
ROWS_PER_W = B_TOTAL // NW
CHUNK = 800
CHUNKS_PER_W = ROWS_PER_W // CHUNK
IDX_W = 100
IDX_ROWS = CHUNK // IDX_W


def _pe_table():
    position = jnp.arange(0, SEQ, dtype=jnp.float32)[:, None]
    div_term = jnp.exp(
        jnp.arange(0, EMBED_DIM, 2, dtype=jnp.float32)
        * (-(math.log(10000.0) / EMBED_DIM)))
    pe = jnp.zeros((SEQ, EMBED_DIM), dtype=jnp.float32)
    pe = pe.at[:, 0::2].set(jnp.sin(position * div_term))
    pe = pe.at[:, 1::2].set(jnp.cos(position * div_term))
    return pe


def _sc_gather_add(x2d, emb_weight, pe):
    mesh = plsc.VectorSubcoreMesh(core_axis_name="c", subcore_axis_name="s")

    @functools.partial(
        pl.kernel,
        out_type=jax.ShapeDtypeStruct((B_TOTAL, EMBED_DIM), jnp.float32),
        mesh=mesh,
        scratch_types=[
            pltpu.VMEM((IDX_ROWS, IDX_W), jnp.int32),
            pltpu.VMEM((CHUNK, EMBED_DIM), jnp.float32),
            pltpu.VMEM((SEQ, EMBED_DIM), jnp.float32),
            pltpu.SemaphoreType.DMA,
        ],
        compiler_params=pltpu.CompilerParams(use_tc_tiling_on_sc=False),
    )
    def k(x_hbm, tab_hbm, pe_hbm, out_hbm, idx_v, rows_v, pe_v, gsem):
        cid = lax.axis_index("c")
        sid = lax.axis_index("s")
        wid = sid * NC + cid
        pltpu.sync_copy(pe_hbm, pe_v)

        def chunk_body(i, carry):
            base = pl.multiple_of((wid * CHUNKS_PER_W + i) * CHUNK, CHUNK)
            irow = pl.multiple_of(base // IDX_W, IDX_ROWS)
            pltpu.sync_copy(x_hbm.at[pl.ds(irow, IDX_ROWS)], idx_v)
            cps = [
                pltpu.async_copy(
                    tab_hbm.at[idx_v.at[g]],
                    rows_v.at[pl.ds(g * IDX_W, IDX_W)],
                    gsem,
                )
                for g in range(IDX_ROWS)
            ]
            for cp in cps:
                cp.wait()

            def add_row(r, c2):
                for s in range(CHUNK // SEQ):
                    for c in range(EMBED_DIM // 16):
                        v = pe_v[r, pl.ds(c * 16, 16)]
                        plsc.addupdate(
                            rows_v.at[s * SEQ + r, pl.ds(c * 16, 16)], v)
                return c2

            lax.fori_loop(0, SEQ, add_row, 0)
            pltpu.sync_copy(rows_v, out_hbm.at[pl.ds(base, CHUNK)])
            return carry

        lax.fori_loop(0, CHUNKS_PER_W, chunk_body, 0)

    return k(x2d, emb_weight, pe)


def _tc_broadcast_pe(pe_flat):
    BS = 128
    D = SEQ * EMBED_DIM

    def body(pe_ref, o_ref):
        o_ref[...] = jnp.broadcast_to(pe_ref[...][None, :], (BS, D))

    return pl.pallas_call(
        body,
        grid=(BATCH // BS,),
        in_specs=[pl.BlockSpec((D,), lambda i: (0,))],
        out_specs=pl.BlockSpec((BS, D), lambda i: (i, 0)),
        out_shape=jax.ShapeDtypeStruct((BATCH, D), jnp.float32),
    )(pe_flat)


def kernel(x, emb_weight):
    pe = _pe_table()
    x2d = x.reshape(B_TOTAL // IDX_W, IDX_W).astype(jnp.int32)
    out = _sc_gather_add(x2d, emb_weight, pe)
    pos_emb = _tc_broadcast_pe(pe.reshape(-1))
    return (
        out.reshape(BATCH, SEQ, EMBED_DIM),
        pos_emb.reshape(BATCH, SEQ, EMBED_DIM),
    )


# speedup vs baseline: 2.4918x; 1.1105x over previous
import math
import functools

import jax
import jax.numpy as jnp
from jax import lax
from jax.experimental import pallas as pl
from jax.experimental.pallas import tpu as pltpu
from jax.experimental.pallas import tpu_sc as plsc

NUM_EMBEDDINGS = 1000000
EMBED_DIM = 64
SEQ = 200
BATCH = 4096

NC = 2
NS = 16
NW = NC * NS

B_TOTAL = BATCH * SEQ
ROWS_PER_W = B_TOTAL // NW
CHUNK = 800
CHUNKS_PER_W = ROWS_PER_W // CHUNK
IDX_W = 100
IDX_ROWS = CHUNK // IDX_W


def _pe_table():
    position = jnp.arange(0, SEQ, dtype=jnp.float32)[:, None]
    div_term = jnp.exp(
        jnp.arange(0, EMBED_DIM, 2, dtype=jnp.float32)
        * (-(math.log(10000.0) / EMBED_DIM)))
    pe = jnp.zeros((SEQ, EMBED_DIM), dtype=jnp.float32)
    pe = pe.at[:, 0::2].set(jnp.sin(position * div_term))
    pe = pe.at[:, 1::2].set(jnp.cos(position * div_term))
    return pe


def _sc_gather_add(x2d, emb_weight, pe):
    mesh = plsc.VectorSubcoreMesh(core_axis_name="c", subcore_axis_name="s")

    @functools.partial(
        pl.kernel,
        out_type=jax.ShapeDtypeStruct((B_TOTAL, EMBED_DIM), jnp.float32),
        mesh=mesh,
        scratch_types=[
            pltpu.VMEM((IDX_ROWS, IDX_W), jnp.int32),
            pltpu.VMEM((CHUNK, EMBED_DIM), jnp.float32),
            pltpu.VMEM((SEQ, EMBED_DIM), jnp.float32),
            pltpu.SemaphoreType.DMA,
        ],
        compiler_params=pltpu.CompilerParams(use_tc_tiling_on_sc=False),
    )
    def k(x_hbm, tab_hbm, pe_hbm, out_hbm, idx_v, rows_v, pe_v, gsem):
        cid = lax.axis_index("c")
        sid = lax.axis_index("s")
        wid = sid * NC + cid
        pltpu.sync_copy(pe_hbm, pe_v)

        def chunk_body(i, carry):
            base = pl.multiple_of((wid * CHUNKS_PER_W + i) * CHUNK, CHUNK)
            irow = pl.multiple_of(base // IDX_W, IDX_ROWS)
            pltpu.sync_copy(x_hbm.at[pl.ds(irow, IDX_ROWS)], idx_v)
            cps = [
                pltpu.async_copy(
                    tab_hbm.at[idx_v.at[g]],
                    rows_v.at[pl.ds(g * IDX_W, IDX_W)],
                    gsem,
                )
                for g in range(IDX_ROWS)
            ]
            for cp in cps:
                cp.wait()

            def add_row(r, c2):
                for s in range(CHUNK // SEQ):
                    for c in range(EMBED_DIM // 16):
                        v = pe_v[r, pl.ds(c * 16, 16)]
                        plsc.addupdate(
                            rows_v.at[s * SEQ + r, pl.ds(c * 16, 16)], v)
                return c2

            lax.fori_loop(0, SEQ, add_row, 0)
            pltpu.sync_copy(rows_v, out_hbm.at[pl.ds(base, CHUNK)])
            return carry

        lax.fori_loop(0, CHUNKS_PER_W, chunk_body, 0)

    return k(x2d, emb_weight, pe)


def kernel(x, emb_weight):
    pe = _pe_table()
    x2d = x.reshape(B_TOTAL // IDX_W, IDX_W).astype(jnp.int32)
    out = _sc_gather_add(x2d, emb_weight, pe)
    pos_emb = jnp.broadcast_to(pe[None, :, :], (BATCH, SEQ, EMBED_DIM))
    return (
        out.reshape(BATCH, SEQ, EMBED_DIM),
        pos_emb,
    )
